# baseline (speedup 1.0000x reference)
import functools

import jax
import jax.numpy as jnp
from jax.experimental import pallas as pl
from jax.experimental.pallas import tpu as pltpu

OUT_LANES = 128
VMEM_LIMIT = 60 * 1024 * 1024
NC = 8
NW = 4


def _l2_normalize(x, eps=1e-12):
    ss = jnp.sum(x * x, axis=-1, keepdims=True)
    return x * jax.lax.rsqrt(jnp.maximum(ss, eps * eps))


def _fused_kernel(t_ref, lm_ref, erow_ref, emb_ref, z3_ref, w1_ref, b1_ref,
                  w2_ref, b2_ref, logtau_ref, out_ref,
                  ev_acc, cens_slots, enc_acc, h_acc, h_slots, mask_s, stat_s,
                  *, M, B, H, RCE, RC1, RC2):
    k = pl.program_id(0)

    @pl.when(k == 0)
    def _():
        t = t_ref[...]
        lm = lm_ref[...]
        cnt = jnp.sum((lm < t).astype(jnp.int32), axis=1, keepdims=True)
        indx = jnp.where(cnt == 0, 1, cnt)
        indx = jnp.where(indx == M, M - 1, indx)
        kf = jax.lax.broadcasted_iota(jnp.int32, (B, M), 1)
        oh_i = (kf == indx).astype(jnp.float32)
        oh_im1 = (kf == (indx - 1)).astype(jnp.float32)
        lm_i = jnp.sum(oh_i * lm, axis=1, keepdims=True)
        lm_im1 = jnp.sum(oh_im1 * lm, axis=1, keepdims=True)
        s = (t - lm_im1) / (lm_i - lm_im1)
        mask_s[...] = (kf >= indx).astype(jnp.float32)
        stat = jnp.concatenate(
            [indx.astype(jnp.float32), s] + [s] * (OUT_LANES - 2), axis=1)
        stat_s[...] = stat

    indx_f = stat_s[:, 0:1]
    s = stat_s[:, 1:2]

    emb = emb_ref[...]

    kloc = (jax.lax.broadcasted_iota(jnp.int32, (B, RCE), 1).astype(jnp.float32)
            + (k * RCE).astype(jnp.float32))
    w_ev = ((kloc == (indx_f - 1.0)).astype(jnp.float32) * (1.0 - s)
            + (kloc == indx_f).astype(jnp.float32) * s)
    evc = jax.lax.dot_general(w_ev, emb, (((1,), (0,)), ((), ())),
                              preferred_element_type=jnp.float32)

    cens_slots[k] = jax.lax.dot_general(mask_s[...], emb, (((1,), (1,)), ((), ())),
                                        preferred_element_type=jnp.float32)

    @pl.when(k == 0)
    def _():
        ev_acc[...] = evc

    @pl.when(k > 0)
    def _():
        ev_acc[...] = ev_acc[...] + evc

    @pl.when(k < NW)
    def _():
        hc = jax.lax.dot_general(z3_ref[0], w1_ref[...], (((1,), (0,)), ((), ())),
                                 preferred_element_type=jnp.float32)

        @pl.when(k == 0)
        def _():
            h_acc[...] = hc

        @pl.when(k > 0)
        def _():
            h_acc[...] = h_acc[...] + hc

    @pl.when(k == NW)
    def _():
        h = jnp.maximum(h_acc[...] + b1_ref[...], 0.0)
        for j in range(NW):
            h_slots[j] = h[:, j * RC2:(j + 1) * RC2]

    @pl.when(k >= NW)
    def _():
        encc = jax.lax.dot_general(h_slots[k - NW], w2_ref[...],
                                   (((1,), (0,)), ((), ())),
                                   preferred_element_type=jnp.float32)

        @pl.when(k == NW)
        def _():
            enc_acc[...] = encc

        @pl.when(k > NW)
        def _():
            enc_acc[...] = enc_acc[...] + encc

    @pl.when(k == NC - 1)
    def _():
        eye_f = (jax.lax.broadcasted_iota(jnp.int32, (B, B), 0)
                 == jax.lax.broadcasted_iota(jnp.int32, (B, B), 1)
                 ).astype(jnp.float32)

        def to_row(col):
            return jax.lax.dot_general(col, eye_f, (((0,), (0,)), ((), ())),
                                       preferred_element_type=jnp.float32)

        ev = ev_acc[...]
        ssev_row = to_row(jnp.sum(ev * ev, axis=1, keepdims=True))
        tinv_row = to_row(1.0 / (jnp.float32(M) - indx_f))

        sscn = jnp.zeros((B, 1), jnp.float32)
        for j in range(NC):
            cj = cens_slots[j]
            sscn = sscn + jnp.sum(cj * cj, axis=1, keepdims=True)
        sscn_row = to_row(sscn)

        inv_tau_sq = jnp.exp(-logtau_ref[...])
        enc = enc_acc[...] + b2_ref[...]
        hz = _l2_normalize(enc) * inv_tau_sq

        sim_ev = jax.lax.dot_general(hz, ev, (((1,), (1,)), ((), ())),
                                     preferred_element_type=jnp.float32)
        sim_cn = jnp.zeros((B, B), jnp.float32)
        for j in range(NC):
            sim_cn = sim_cn + jax.lax.dot_general(
                hz[:, j * RCE:(j + 1) * RCE], cens_slots[j],
                (((1,), (1,)), ((), ())), preferred_element_type=jnp.float32)

        e_row = erow_ref[...]
        w_cn = (1.0 - e_row) * tinv_row
        ss_row = e_row * ssev_row + w_cn * tinv_row * sscn_row
        scale = jax.lax.rsqrt(jnp.maximum(ss_row, jnp.float32(1e-24)))
        sim = (e_row * sim_ev + w_cn * sim_cn) * scale

        g = jnp.sum(sim * eye_f, axis=1, keepdims=True)
        mx = jnp.max(sim, axis=1, keepdims=True)
        lse = mx + jnp.log(jnp.sum(jnp.exp(sim - mx), axis=1, keepdims=True))
        out = jnp.clip((lse - g) - jnp.log(jnp.float32(B)), -5.0, 15.0)
        out_ref[...] = jnp.broadcast_to(out, out_ref.shape)


def kernel(z, t, e, time_landmark, time_emb, w1, b1, w2, b2, log_tau):
    B, M = z.shape
    H = w1.shape[1]
    RCE = M // NC
    RC1 = M // NW
    RC2 = H // NW

    t2 = jnp.asarray(t).reshape(B, 1).astype(jnp.float32)
    erow = jnp.asarray(e).reshape(1, B).astype(jnp.float32)
    lm2 = jnp.asarray(time_landmark).reshape(1, M).astype(jnp.float32)
    emb = jnp.asarray(time_emb).astype(jnp.float32)
    zf = jnp.asarray(z).astype(jnp.float32)
    z3 = jnp.transpose(zf.reshape(B, NW, RC1), (1, 0, 2))
    w1f = jnp.asarray(w1).astype(jnp.float32)
    w2f = jnp.asarray(w2).astype(jnp.float32)
    b1f = jnp.asarray(b1).reshape(1, H).astype(jnp.float32)
    b2f = jnp.asarray(b2).reshape(1, M).astype(jnp.float32)
    logtau2 = jnp.asarray(log_tau).reshape(1, 1).astype(jnp.float32)

    out_wide = pl.pallas_call(
        functools.partial(_fused_kernel, M=M, B=B, H=H,
                          RCE=RCE, RC1=RC1, RC2=RC2),
        out_shape=jax.ShapeDtypeStruct((B, OUT_LANES), jnp.float32),
        grid=(NC,),
        in_specs=[
            pl.BlockSpec((B, 1), lambda k: (0, 0)),
            pl.BlockSpec((1, M), lambda k: (0, 0)),
            pl.BlockSpec((1, B), lambda k: (0, 0)),
            pl.BlockSpec((RCE, M), lambda k: (k, 0)),
            pl.BlockSpec((1, B, RC1),
                         lambda k: (jnp.minimum(k, NW - 1), 0, 0)),
            pl.BlockSpec((RC1, H),
                         lambda k: (jnp.minimum(k, NW - 1), 0)),
            pl.BlockSpec((1, H), lambda k: (0, 0)),
            pl.BlockSpec((RC2, M),
                         lambda k: (jnp.clip(k - NW, 0, NW - 1), 0)),
            pl.BlockSpec((1, M), lambda k: (0, 0)),
            pl.BlockSpec((1, 1), lambda k: (0, 0)),
        ],
        out_specs=pl.BlockSpec((B, OUT_LANES), lambda k: (0, 0)),
        scratch_shapes=[
            pltpu.VMEM((B, M), jnp.float32),
            pltpu.VMEM((NC, B, RCE), jnp.float32),
            pltpu.VMEM((B, M), jnp.float32),
            pltpu.VMEM((B, H), jnp.float32),
            pltpu.VMEM((NW, B, RC2), jnp.float32),
            pltpu.VMEM((B, M), jnp.float32),
            pltpu.VMEM((B, OUT_LANES), jnp.float32),
        ],
        compiler_params=pltpu.CompilerParams(
            dimension_semantics=("arbitrary",),
            vmem_limit_bytes=VMEM_LIMIT),
        cost_estimate=pl.CostEstimate(
            flops=int(6 * B * M * M // NC + 4 * B * M * H // NC),
            transcendentals=int(B * B + 4 * B),
            bytes_accessed=int(4 * (M * M + M * H + H * M + B * M) // NC),
    ))(t2, lm2, erow, emb, z3, w1f, b1f, w2f, b2f, logtau2)

    return out_wide[:, :1]


# speedup vs baseline: 1.0844x; 1.0844x over previous
import functools

import jax
import jax.numpy as jnp
from jax.experimental import pallas as pl
from jax.experimental.pallas import tpu as pltpu

OUT_LANES = 128
VMEM_LIMIT = 60 * 1024 * 1024
NC = 8
NW = 4


def _l2_normalize(x, eps=1e-12):
    ss = jnp.sum(x * x, axis=-1, keepdims=True)
    return x * jax.lax.rsqrt(jnp.maximum(ss, eps * eps))


def _fused_kernel(t_ref, lm_ref, erow_ref, emb_ref, z3_ref, w1_ref, b1_ref,
                  w2_ref, b2_ref, logtau_ref, out_ref,
                  ev_acc, cens_slots, enc_acc, h_acc, h_slots, mask_s, stat_s,
                  sval_s, *, M, B, H, RCE, RC1, RC2):
    k = pl.program_id(0)

    @pl.when(k == 0)
    def _():
        t = t_ref[...]
        lm = lm_ref[...]
        cnt = jnp.sum((lm < t).astype(jnp.int32), axis=1, keepdims=True)
        indx = jnp.where(cnt == 0, 1, cnt)
        indx = jnp.where(indx == M, M - 1, indx)
        kf = jax.lax.broadcasted_iota(jnp.int32, (B, M), 1)
        oh_i = (kf == indx).astype(jnp.float32)
        oh_im1 = (kf == (indx - 1)).astype(jnp.float32)
        lm_i = jnp.sum(oh_i * lm, axis=1, keepdims=True)
        lm_im1 = jnp.sum(oh_im1 * lm, axis=1, keepdims=True)
        s = (t - lm_im1) / (lm_i - lm_im1)
        mask_s[...] = (kf >= indx).astype(jnp.float32)
        stat_s[...] = jnp.broadcast_to(indx.astype(jnp.float32),
                                       (B, OUT_LANES))
        sval_s[...] = jnp.broadcast_to(s, (B, OUT_LANES))

    indx_f = stat_s[:, 0:1]
    s = sval_s[:, 0:1]

    emb = emb_ref[...]

    kloc = (jax.lax.broadcasted_iota(jnp.int32, (B, RCE), 1).astype(jnp.float32)
            + (k * RCE).astype(jnp.float32))
    w_ev = ((kloc == (indx_f - 1.0)).astype(jnp.float32) * (1.0 - s)
            + (kloc == indx_f).astype(jnp.float32) * s)
    evc = jax.lax.dot_general(w_ev, emb, (((1,), (0,)), ((), ())),
                              preferred_element_type=jnp.float32)

    cens_slots[k] = jax.lax.dot_general(mask_s[...], emb, (((1,), (1,)), ((), ())),
                                        preferred_element_type=jnp.float32)

    @pl.when(k == 0)
    def _():
        ev_acc[...] = evc

    @pl.when(k > 0)
    def _():
        ev_acc[...] = ev_acc[...] + evc

    @pl.when(k < NW)
    def _():
        hc = jax.lax.dot_general(z3_ref[0], w1_ref[...], (((1,), (0,)), ((), ())),
                                 preferred_element_type=jnp.float32)

        @pl.when(k == 0)
        def _():
            h_acc[...] = hc

        @pl.when(k > 0)
        def _():
            h_acc[...] = h_acc[...] + hc

    @pl.when(k == NW)
    def _():
        h = jnp.maximum(h_acc[...] + b1_ref[...], 0.0)
        for j in range(NW):
            h_slots[j] = h[:, j * RC2:(j + 1) * RC2]

    @pl.when(k >= NW)
    def _():
        encc = jax.lax.dot_general(h_slots[k - NW], w2_ref[...],
                                   (((1,), (0,)), ((), ())),
                                   preferred_element_type=jnp.float32)

        @pl.when(k == NW)
        def _():
            enc_acc[...] = encc

        @pl.when(k > NW)
        def _():
            enc_acc[...] = enc_acc[...] + encc

    @pl.when(k == NC - 1)
    def _():
        eye_f = (jax.lax.broadcasted_iota(jnp.int32, (B, B), 0)
                 == jax.lax.broadcasted_iota(jnp.int32, (B, B), 1)
                 ).astype(jnp.float32)

        def to_row(col):
            return jax.lax.dot_general(col, eye_f, (((0,), (0,)), ((), ())),
                                       preferred_element_type=jnp.float32)

        ev = ev_acc[...]
        ssev_row = to_row(jnp.sum(ev * ev, axis=1, keepdims=True))
        tinv_row = to_row(1.0 / (jnp.float32(M) - indx_f))

        sscn = jnp.zeros((B, 1), jnp.float32)
        for j in range(NC):
            cj = cens_slots[j]
            sscn = sscn + jnp.sum(cj * cj, axis=1, keepdims=True)
        sscn_row = to_row(sscn)

        inv_tau_sq = jnp.exp(-logtau_ref[...])
        enc = enc_acc[...] + b2_ref[...]
        hz = _l2_normalize(enc) * inv_tau_sq

        sim_ev = jax.lax.dot_general(hz, ev, (((1,), (1,)), ((), ())),
                                     preferred_element_type=jnp.float32)
        sim_cn = jnp.zeros((B, B), jnp.float32)
        for j in range(NC):
            sim_cn = sim_cn + jax.lax.dot_general(
                hz[:, j * RCE:(j + 1) * RCE], cens_slots[j],
                (((1,), (1,)), ((), ())), preferred_element_type=jnp.float32)

        e_row = erow_ref[...]
        w_cn = (1.0 - e_row) * tinv_row
        ss_row = e_row * ssev_row + w_cn * tinv_row * sscn_row
        scale = jax.lax.rsqrt(jnp.maximum(ss_row, jnp.float32(1e-24)))
        sim = (e_row * sim_ev + w_cn * sim_cn) * scale

        g = jnp.sum(sim * eye_f, axis=1, keepdims=True)
        mx = jnp.max(sim, axis=1, keepdims=True)
        lse = mx + jnp.log(jnp.sum(jnp.exp(sim - mx), axis=1, keepdims=True))
        out = jnp.clip((lse - g) - jnp.log(jnp.float32(B)), -5.0, 15.0)
        out_ref[...] = jnp.broadcast_to(out, out_ref.shape)


def kernel(z, t, e, time_landmark, time_emb, w1, b1, w2, b2, log_tau):
    B, M = z.shape
    H = w1.shape[1]
    RCE = M // NC
    RC1 = M // NW
    RC2 = H // NW

    t2 = jnp.asarray(t).reshape(B, 1).astype(jnp.float32)
    erow = jnp.asarray(e).reshape(1, B).astype(jnp.float32)
    lm2 = jnp.asarray(time_landmark).reshape(1, M).astype(jnp.float32)
    emb = jnp.asarray(time_emb).astype(jnp.float32)
    zf = jnp.asarray(z).astype(jnp.float32)
    z3 = jnp.transpose(zf.reshape(B, NW, RC1), (1, 0, 2))
    w1f = jnp.asarray(w1).astype(jnp.float32)
    w2f = jnp.asarray(w2).astype(jnp.float32)
    b1f = jnp.asarray(b1).reshape(1, H).astype(jnp.float32)
    b2f = jnp.asarray(b2).reshape(1, M).astype(jnp.float32)
    logtau2 = jnp.asarray(log_tau).reshape(1, 1).astype(jnp.float32)

    out_wide = pl.pallas_call(
        functools.partial(_fused_kernel, M=M, B=B, H=H,
                          RCE=RCE, RC1=RC1, RC2=RC2),
        out_shape=jax.ShapeDtypeStruct((B, OUT_LANES), jnp.float32),
        grid=(NC,),
        in_specs=[
            pl.BlockSpec((B, 1), lambda k: (0, 0)),
            pl.BlockSpec((1, M), lambda k: (0, 0)),
            pl.BlockSpec((1, B), lambda k: (0, 0)),
            pl.BlockSpec((RCE, M), lambda k: (k, 0)),
            pl.BlockSpec((1, B, RC1),
                         lambda k: (jnp.minimum(k, NW - 1), 0, 0)),
            pl.BlockSpec((RC1, H),
                         lambda k: (jnp.minimum(k, NW - 1), 0)),
            pl.BlockSpec((1, H), lambda k: (0, 0)),
            pl.BlockSpec((RC2, M),
                         lambda k: (jnp.clip(k - NW, 0, NW - 1), 0)),
            pl.BlockSpec((1, M), lambda k: (0, 0)),
            pl.BlockSpec((1, 1), lambda k: (0, 0)),
        ],
        out_specs=pl.BlockSpec((B, OUT_LANES), lambda k: (0, 0)),
        scratch_shapes=[
            pltpu.VMEM((B, M), jnp.float32),
            pltpu.VMEM((NC, B, RCE), jnp.float32),
            pltpu.VMEM((B, M), jnp.float32),
            pltpu.VMEM((B, H), jnp.float32),
            pltpu.VMEM((NW, B, RC2), jnp.float32),
            pltpu.VMEM((B, M), jnp.float32),
            pltpu.VMEM((B, OUT_LANES), jnp.float32),
            pltpu.VMEM((B, OUT_LANES), jnp.float32),
        ],
        compiler_params=pltpu.CompilerParams(
            dimension_semantics=("arbitrary",),
            vmem_limit_bytes=VMEM_LIMIT),
        cost_estimate=pl.CostEstimate(
            flops=int(6 * B * M * M // NC + 4 * B * M * H // NC),
            transcendentals=int(B * B + 4 * B),
            bytes_accessed=int(4 * (M * M + M * H + H * M + B * M) // NC),
    ))(t2, lm2, erow, emb, z3, w1f, b1f, w2f, b2f, logtau2)

    return out_wide[:, :1]


# speedup vs baseline: 1.0845x; 1.0001x over previous
import functools

import jax
import jax.numpy as jnp
from jax.experimental import pallas as pl
from jax.experimental.pallas import tpu as pltpu

OUT_LANES = 128
VMEM_LIMIT = 60 * 1024 * 1024
NC = 8
NW = 4


def _l2_normalize(x, eps=1e-12):
    ss = jnp.sum(x * x, axis=-1, keepdims=True)
    return x * jax.lax.rsqrt(jnp.maximum(ss, eps * eps))


def _fused_kernel(t_ref, lm_ref, erow_ref, emb_ref, z3_ref, w1_ref, b1_ref,
                  w2_ref, b2_ref, logtau_ref, out_ref,
                  ev_acc, cens_slots, enc_acc, h_acc, h_slots, mask_s, stat_s,
                  sval_s, *, M, B, H, RCE, RC1, RC2):
    k = pl.program_id(0)

    @pl.when(k == 0)
    def _():
        t = t_ref[...]
        lm = lm_ref[...]
        cnt = jnp.sum((lm < t).astype(jnp.int32), axis=1, keepdims=True)
        indx = jnp.where(cnt == 0, 1, cnt)
        indx = jnp.where(indx == M, M - 1, indx)
        kf = jax.lax.broadcasted_iota(jnp.int32, (B, M), 1)
        oh_i = (kf == indx).astype(jnp.float32)
        oh_im1 = (kf == (indx - 1)).astype(jnp.float32)
        lm_i = jnp.sum(oh_i * lm, axis=1, keepdims=True)
        lm_im1 = jnp.sum(oh_im1 * lm, axis=1, keepdims=True)
        s = (t - lm_im1) / (lm_i - lm_im1)
        mask_s[...] = (kf >= indx).astype(jnp.float32)
        stat_s[...] = jnp.broadcast_to(indx.astype(jnp.float32),
                                       (B, OUT_LANES))
        sval_s[...] = jnp.broadcast_to(s, (B, OUT_LANES))

    indx_f = stat_s[:, 0:1]
    s = sval_s[:, 0:1]

    emb_bf = emb_ref[...].astype(jnp.bfloat16)

    kloc = (jax.lax.broadcasted_iota(jnp.int32, (B, RCE), 1).astype(jnp.float32)
            + (k * RCE).astype(jnp.float32))
    w_ev = ((kloc == (indx_f - 1.0)).astype(jnp.float32) * (1.0 - s)
            + (kloc == indx_f).astype(jnp.float32) * s).astype(jnp.bfloat16)
    evc = jax.lax.dot_general(w_ev, emb_bf, (((1,), (0,)), ((), ())),
                              preferred_element_type=jnp.float32)

    cens_slots[k] = jax.lax.dot_general(mask_s[...].astype(jnp.bfloat16),
                                        emb_bf, (((1,), (1,)), ((), ())),
                                        preferred_element_type=jnp.float32)

    @pl.when(k == 0)
    def _():
        ev_acc[...] = evc

    @pl.when(k > 0)
    def _():
        ev_acc[...] = ev_acc[...] + evc

    @pl.when(k < NW)
    def _():
        hc = jax.lax.dot_general(z3_ref[0], w1_ref[...], (((1,), (0,)), ((), ())),
                                 preferred_element_type=jnp.float32)

        @pl.when(k == 0)
        def _():
            h_acc[...] = hc

        @pl.when(k > 0)
        def _():
            h_acc[...] = h_acc[...] + hc

    @pl.when(k == NW)
    def _():
        h = jnp.maximum(h_acc[...] + b1_ref[...], 0.0)
        for j in range(NW):
            h_slots[j] = h[:, j * RC2:(j + 1) * RC2]

    @pl.when(k >= NW)
    def _():
        encc = jax.lax.dot_general(h_slots[k - NW], w2_ref[...],
                                   (((1,), (0,)), ((), ())),
                                   preferred_element_type=jnp.float32)

        @pl.when(k == NW)
        def _():
            enc_acc[...] = encc

        @pl.when(k > NW)
        def _():
            enc_acc[...] = enc_acc[...] + encc

    @pl.when(k == NC - 1)
    def _():
        eye_f = (jax.lax.broadcasted_iota(jnp.int32, (B, B), 0)
                 == jax.lax.broadcasted_iota(jnp.int32, (B, B), 1)
                 ).astype(jnp.float32)

        def to_row(col):
            return jax.lax.dot_general(col, eye_f, (((0,), (0,)), ((), ())),
                                       preferred_element_type=jnp.float32)

        ev = ev_acc[...]
        ssev_row = to_row(jnp.sum(ev * ev, axis=1, keepdims=True))
        tinv_row = to_row(1.0 / (jnp.float32(M) - indx_f))

        sscn = jnp.zeros((B, 1), jnp.float32)
        for j in range(NC):
            cj = cens_slots[j]
            sscn = sscn + jnp.sum(cj * cj, axis=1, keepdims=True)
        sscn_row = to_row(sscn)

        inv_tau_sq = jnp.exp(-logtau_ref[...])
        enc = enc_acc[...] + b2_ref[...]
        hz = _l2_normalize(enc) * inv_tau_sq

        sim_ev = jax.lax.dot_general(hz, ev, (((1,), (1,)), ((), ())),
                                     preferred_element_type=jnp.float32)
        sim_cn = jnp.zeros((B, B), jnp.float32)
        for j in range(NC):
            sim_cn = sim_cn + jax.lax.dot_general(
                hz[:, j * RCE:(j + 1) * RCE], cens_slots[j],
                (((1,), (1,)), ((), ())), preferred_element_type=jnp.float32)

        e_row = erow_ref[...]
        w_cn = (1.0 - e_row) * tinv_row
        ss_row = e_row * ssev_row + w_cn * tinv_row * sscn_row
        scale = jax.lax.rsqrt(jnp.maximum(ss_row, jnp.float32(1e-24)))
        sim = (e_row * sim_ev + w_cn * sim_cn) * scale

        g = jnp.sum(sim * eye_f, axis=1, keepdims=True)
        mx = jnp.max(sim, axis=1, keepdims=True)
        lse = mx + jnp.log(jnp.sum(jnp.exp(sim - mx), axis=1, keepdims=True))
        out = jnp.clip((lse - g) - jnp.log(jnp.float32(B)), -5.0, 15.0)
        out_ref[...] = jnp.broadcast_to(out, out_ref.shape)


def kernel(z, t, e, time_landmark, time_emb, w1, b1, w2, b2, log_tau):
    B, M = z.shape
    H = w1.shape[1]
    RCE = M // NC
    RC1 = M // NW
    RC2 = H // NW

    t2 = jnp.asarray(t).reshape(B, 1).astype(jnp.float32)
    erow = jnp.asarray(e).reshape(1, B).astype(jnp.float32)
    lm2 = jnp.asarray(time_landmark).reshape(1, M).astype(jnp.float32)
    emb = jnp.asarray(time_emb).astype(jnp.float32)
    zf = jnp.asarray(z).astype(jnp.float32)
    z3 = jnp.transpose(zf.reshape(B, NW, RC1), (1, 0, 2))
    w1f = jnp.asarray(w1).astype(jnp.float32)
    w2f = jnp.asarray(w2).astype(jnp.float32)
    b1f = jnp.asarray(b1).reshape(1, H).astype(jnp.float32)
    b2f = jnp.asarray(b2).reshape(1, M).astype(jnp.float32)
    logtau2 = jnp.asarray(log_tau).reshape(1, 1).astype(jnp.float32)

    out_wide = pl.pallas_call(
        functools.partial(_fused_kernel, M=M, B=B, H=H,
                          RCE=RCE, RC1=RC1, RC2=RC2),
        out_shape=jax.ShapeDtypeStruct((B, OUT_LANES), jnp.float32),
        grid=(NC,),
        in_specs=[
            pl.BlockSpec((B, 1), lambda k: (0, 0)),
            pl.BlockSpec((1, M), lambda k: (0, 0)),
            pl.BlockSpec((1, B), lambda k: (0, 0)),
            pl.BlockSpec((RCE, M), lambda k: (k, 0)),
            pl.BlockSpec((1, B, RC1),
                         lambda k: (jnp.minimum(k, NW - 1), 0, 0)),
            pl.BlockSpec((RC1, H),
                         lambda k: (jnp.minimum(k, NW - 1), 0)),
            pl.BlockSpec((1, H), lambda k: (0, 0)),
            pl.BlockSpec((RC2, M),
                         lambda k: (jnp.clip(k - NW, 0, NW - 1), 0)),
            pl.BlockSpec((1, M), lambda k: (0, 0)),
            pl.BlockSpec((1, 1), lambda k: (0, 0)),
        ],
        out_specs=pl.BlockSpec((B, OUT_LANES), lambda k: (0, 0)),
        scratch_shapes=[
            pltpu.VMEM((B, M), jnp.float32),
            pltpu.VMEM((NC, B, RCE), jnp.float32),
            pltpu.VMEM((B, M), jnp.float32),
            pltpu.VMEM((B, H), jnp.float32),
            pltpu.VMEM((NW, B, RC2), jnp.float32),
            pltpu.VMEM((B, M), jnp.float32),
            pltpu.VMEM((B, OUT_LANES), jnp.float32),
            pltpu.VMEM((B, OUT_LANES), jnp.float32),
        ],
        compiler_params=pltpu.CompilerParams(
            dimension_semantics=("arbitrary",),
            vmem_limit_bytes=VMEM_LIMIT),
        cost_estimate=pl.CostEstimate(
            flops=int(6 * B * M * M // NC + 4 * B * M * H // NC),
            transcendentals=int(B * B + 4 * B),
            bytes_accessed=int(4 * (M * M + M * H + H * M + B * M) // NC),
    ))(t2, lm2, erow, emb, z3, w1f, b1f, w2f, b2f, logtau2)

    return out_wide[:, :1]


# speedup vs baseline: 1.2334x; 1.1372x over previous
import functools

import jax
import jax.numpy as jnp
from jax.experimental import pallas as pl
from jax.experimental.pallas import tpu as pltpu

OUT_LANES = 128
VMEM_LIMIT = 60 * 1024 * 1024
NC = 8


def _l2_normalize(x, eps=1e-12):
    ss = jnp.sum(x * x, axis=-1, keepdims=True)
    return x * jax.lax.rsqrt(jnp.maximum(ss, eps * eps))


def _fused_kernel(t_ref, lm_ref, erow_ref, emb_ref, z_ref, w1_ref, b1_ref,
                  w2_ref, b2_ref, logtau_ref, out_ref,
                  ev_acc, cens_slots, enc_acc, mask_s, indx_s, sval_s,
                  *, M, B, RCE):
    k = pl.program_id(0)

    @pl.when(k == 0)
    def _():
        t = t_ref[...]
        lm = lm_ref[...]
        cnt = jnp.sum((lm < t).astype(jnp.int32), axis=1, keepdims=True)
        indx = jnp.where(cnt == 0, 1, cnt)
        indx = jnp.where(indx == M, M - 1, indx)
        kf = jax.lax.broadcasted_iota(jnp.int32, (B, M), 1)
        oh_i = (kf == indx).astype(jnp.float32)
        oh_im1 = (kf == (indx - 1)).astype(jnp.float32)
        lm_i = jnp.sum(oh_i * lm, axis=1, keepdims=True)
        lm_im1 = jnp.sum(oh_im1 * lm, axis=1, keepdims=True)
        s = (t - lm_im1) / (lm_i - lm_im1)
        mask_s[...] = (kf >= indx).astype(jnp.bfloat16)
        indx_s[...] = jnp.broadcast_to(indx.astype(jnp.float32),
                                       (B, OUT_LANES))
        sval_s[...] = jnp.broadcast_to(s, (B, OUT_LANES))

    indx_f = indx_s[:, 0:1]
    s = sval_s[:, 0:1]

    emb_bf = emb_ref[...].astype(jnp.bfloat16)

    kloc = (jax.lax.broadcasted_iota(jnp.int32, (B, RCE), 1).astype(jnp.float32)
            + (k * RCE).astype(jnp.float32))
    w_ev = ((kloc == (indx_f - 1.0)).astype(jnp.float32) * (1.0 - s)
            + (kloc == indx_f).astype(jnp.float32) * s).astype(jnp.bfloat16)
    evc = jax.lax.dot_general(w_ev, emb_bf, (((1,), (0,)), ((), ())),
                              preferred_element_type=jnp.float32)

    cens_slots[k] = jax.lax.dot_general(mask_s[...], emb_bf, (((1,), (1,)), ((), ())),
                                        preferred_element_type=jnp.float32)

    h = jnp.maximum(
        jnp.dot(z_ref[...].astype(jnp.bfloat16), w1_ref[...].astype(jnp.bfloat16),
                preferred_element_type=jnp.float32)
        + b1_ref[...], 0.0)
    encc = jnp.dot(h.astype(jnp.bfloat16), w2_ref[...].astype(jnp.bfloat16),
                   preferred_element_type=jnp.float32)

    @pl.when(k == 0)
    def _():
        ev_acc[...] = evc
        enc_acc[...] = encc

    @pl.when(k > 0)
    def _():
        ev_acc[...] = ev_acc[...] + evc
        enc_acc[...] = enc_acc[...] + encc

    @pl.when(k == NC - 1)
    def _():
        eye_f = (jax.lax.broadcasted_iota(jnp.int32, (B, B), 0)
                 == jax.lax.broadcasted_iota(jnp.int32, (B, B), 1)
                 ).astype(jnp.float32)

        def to_row(col):
            return jax.lax.dot_general(col, eye_f, (((0,), (0,)), ((), ())),
                                       preferred_element_type=jnp.float32)

        ev = ev_acc[...]
        ssev_row = to_row(jnp.sum(ev * ev, axis=1, keepdims=True))
        tinv_row = to_row(1.0 / (jnp.float32(M) - indx_f))

        sscn = jnp.zeros((B, 1), jnp.float32)
        for j in range(NC):
            cj = cens_slots[j]
            sscn = sscn + jnp.sum(cj * cj, axis=1, keepdims=True)
        sscn_row = to_row(sscn)

        inv_tau_sq = jnp.exp(-logtau_ref[...])
        enc = enc_acc[...] + b2_ref[...]
        hz = _l2_normalize(enc) * inv_tau_sq

        sim_ev = jax.lax.dot_general(hz, ev, (((1,), (1,)), ((), ())),
                                     preferred_element_type=jnp.float32)
        sim_cn = jnp.zeros((B, B), jnp.float32)
        for j in range(NC):
            sim_cn = sim_cn + jax.lax.dot_general(
                hz[:, j * RCE:(j + 1) * RCE], cens_slots[j],
                (((1,), (1,)), ((), ())), preferred_element_type=jnp.float32)

        e_row = erow_ref[...]
        w_cn = (1.0 - e_row) * tinv_row
        ss_row = e_row * ssev_row + w_cn * tinv_row * sscn_row
        scale = jax.lax.rsqrt(jnp.maximum(ss_row, jnp.float32(1e-24)))
        sim = (e_row * sim_ev + w_cn * sim_cn) * scale

        g = jnp.sum(sim * eye_f, axis=1, keepdims=True)
        mx = jnp.max(sim, axis=1, keepdims=True)
        lse = mx + jnp.log(jnp.sum(jnp.exp(sim - mx), axis=1, keepdims=True))
        out = jnp.clip((lse - g) - jnp.log(jnp.float32(B)), -5.0, 15.0)
        out_ref[...] = jnp.broadcast_to(out, out_ref.shape)


def kernel(z, t, e, time_landmark, time_emb, w1, b1, w2, b2, log_tau):
    B, M = z.shape
    H = w1.shape[1]
    RCE, RCH = M // NC, H // NC

    t2 = jnp.asarray(t).reshape(B, 1).astype(jnp.float32)
    erow = jnp.asarray(e).reshape(1, B).astype(jnp.float32)
    lm2 = jnp.asarray(time_landmark).reshape(1, M).astype(jnp.float32)
    emb = jnp.asarray(time_emb).astype(jnp.float32)
    w1f = jnp.asarray(w1).astype(jnp.float32)
    w2f = jnp.asarray(w2).astype(jnp.float32)
    b1f = jnp.asarray(b1).reshape(1, H).astype(jnp.float32)
    b2f = jnp.asarray(b2).reshape(1, M).astype(jnp.float32)
    logtau2 = jnp.asarray(log_tau).reshape(1, 1).astype(jnp.float32)

    out_wide = pl.pallas_call(
        functools.partial(_fused_kernel, M=M, B=B, RCE=RCE),
        out_shape=jax.ShapeDtypeStruct((B, OUT_LANES), jnp.float32),
        grid=(NC,),
        in_specs=[
            pl.BlockSpec((B, 1), lambda k: (0, 0)),
            pl.BlockSpec((1, M), lambda k: (0, 0)),
            pl.BlockSpec((1, B), lambda k: (0, 0)),
            pl.BlockSpec((RCE, M), lambda k: (k, 0)),
            pl.BlockSpec((B, M), lambda k: (0, 0)),
            pl.BlockSpec((M, RCH), lambda k: (0, k)),
            pl.BlockSpec((1, RCH), lambda k: (0, k)),
            pl.BlockSpec((RCH, M), lambda k: (k, 0)),
            pl.BlockSpec((1, M), lambda k: (0, 0)),
            pl.BlockSpec((1, 1), lambda k: (0, 0)),
        ],
        out_specs=pl.BlockSpec((B, OUT_LANES), lambda k: (0, 0)),
        scratch_shapes=[
            pltpu.VMEM((B, M), jnp.float32),
            pltpu.VMEM((NC, B, RCE), jnp.float32),
            pltpu.VMEM((B, M), jnp.float32),
            pltpu.VMEM((B, M), jnp.bfloat16),
            pltpu.VMEM((B, OUT_LANES), jnp.float32),
            pltpu.VMEM((B, OUT_LANES), jnp.float32),
        ],
        compiler_params=pltpu.CompilerParams(
            dimension_semantics=("arbitrary",),
            vmem_limit_bytes=VMEM_LIMIT),
        cost_estimate=pl.CostEstimate(
            flops=int(6 * B * M * M // NC + 4 * B * M * H // NC),
            transcendentals=int(B * B + 4 * B),
            bytes_accessed=int(4 * (M * M + M * H + H * M + B * M) // NC),
    ))(t2, lm2, erow, emb, z, w1f, b1f, w2f, b2f, logtau2)

    return out_wide[:, :1]


# speedup vs baseline: 1.2520x; 1.0151x over previous
import functools

import jax
import jax.numpy as jnp
from jax.experimental import pallas as pl
from jax.experimental.pallas import tpu as pltpu

OUT_LANES = 128
VMEM_LIMIT = 60 * 1024 * 1024
NC = 8


def _l2_normalize(x, eps=1e-12):
    ss = jnp.sum(x * x, axis=-1, keepdims=True)
    return x * jax.lax.rsqrt(jnp.maximum(ss, eps * eps))


def _fused_kernel(t_ref, lm_ref, erow_ref, emb_ref, z_ref, w1_ref, b1_ref,
                  w2_ref, b2_ref, logtau_ref, out_ref,
                  ev_acc, cens_slots, enc_acc, mask_s, indx_s, z_bf,
                  wev_slots, *, M, B, RCE):
    k = pl.program_id(0)

    @pl.when(k == 0)
    def _():
        t = t_ref[...]
        lm = lm_ref[...]
        cnt = jnp.sum((lm < t).astype(jnp.int32), axis=1, keepdims=True)
        indx = jnp.where(cnt == 0, 1, cnt)
        indx = jnp.where(indx == M, M - 1, indx)
        kf = jax.lax.broadcasted_iota(jnp.int32, (B, M), 1)
        oh_i = (kf == indx).astype(jnp.float32)
        oh_im1 = (kf == (indx - 1)).astype(jnp.float32)
        lm_i = jnp.sum(oh_i * lm, axis=1, keepdims=True)
        lm_im1 = jnp.sum(oh_im1 * lm, axis=1, keepdims=True)
        s = (t - lm_im1) / (lm_i - lm_im1)
        mask_s[...] = (kf >= indx).astype(jnp.bfloat16)
        indx_s[...] = jnp.broadcast_to(indx.astype(jnp.float32),
                                       (B, OUT_LANES))
        z_bf[...] = z_ref[...].astype(jnp.bfloat16)
        w_ev = (oh_im1 * (1.0 - s) + oh_i * s).astype(jnp.bfloat16)
        for j in range(NC):
            wev_slots[j] = w_ev[:, j * RCE:(j + 1) * RCE]

    emb_bf = emb_ref[...].astype(jnp.bfloat16)

    evc = jax.lax.dot_general(wev_slots[k], emb_bf, (((1,), (0,)), ((), ())),
                              preferred_element_type=jnp.float32)

    cens_slots[k] = jax.lax.dot_general(mask_s[...], emb_bf, (((1,), (1,)), ((), ())),
                                        preferred_element_type=jnp.float32)

    h = jnp.maximum(
        jnp.dot(z_bf[...], w1_ref[...].astype(jnp.bfloat16),
                preferred_element_type=jnp.float32)
        + b1_ref[...], 0.0)
    encc = jnp.dot(h.astype(jnp.bfloat16), w2_ref[...].astype(jnp.bfloat16),
                   preferred_element_type=jnp.float32)

    @pl.when(k == 0)
    def _():
        ev_acc[...] = evc
        enc_acc[...] = encc

    @pl.when(k > 0)
    def _():
        ev_acc[...] = ev_acc[...] + evc
        enc_acc[...] = enc_acc[...] + encc

    @pl.when(k == NC - 1)
    def _():
        eye_f = (jax.lax.broadcasted_iota(jnp.int32, (B, B), 0)
                 == jax.lax.broadcasted_iota(jnp.int32, (B, B), 1)
                 ).astype(jnp.float32)

        def to_row(col):
            return jax.lax.dot_general(col, eye_f, (((0,), (0,)), ((), ())),
                                       preferred_element_type=jnp.float32)

        ev = ev_acc[...]
        ssev_row = to_row(jnp.sum(ev * ev, axis=1, keepdims=True))
        tinv_row = to_row(1.0 / (jnp.float32(M) - indx_s[:, 0:1]))

        sscn = jnp.zeros((B, 1), jnp.float32)
        for j in range(NC):
            cj = cens_slots[j]
            sscn = sscn + jnp.sum(cj * cj, axis=1, keepdims=True)
        sscn_row = to_row(sscn)

        inv_tau_sq = jnp.exp(-logtau_ref[...])
        enc = enc_acc[...] + b2_ref[...]
        hz = _l2_normalize(enc) * inv_tau_sq

        sim_ev = jax.lax.dot_general(hz, ev, (((1,), (1,)), ((), ())),
                                     preferred_element_type=jnp.float32)
        sim_cn = jnp.zeros((B, B), jnp.float32)
        for j in range(NC):
            sim_cn = sim_cn + jax.lax.dot_general(
                hz[:, j * RCE:(j + 1) * RCE], cens_slots[j],
                (((1,), (1,)), ((), ())), preferred_element_type=jnp.float32)

        e_row = erow_ref[...]
        w_cn = (1.0 - e_row) * tinv_row
        ss_row = e_row * ssev_row + w_cn * tinv_row * sscn_row
        scale = jax.lax.rsqrt(jnp.maximum(ss_row, jnp.float32(1e-24)))
        sim = (e_row * sim_ev + w_cn * sim_cn) * scale

        g = jnp.sum(sim * eye_f, axis=1, keepdims=True)
        mx = jnp.max(sim, axis=1, keepdims=True)
        lse = mx + jnp.log(jnp.sum(jnp.exp(sim - mx), axis=1, keepdims=True))
        out = jnp.clip((lse - g) - jnp.log(jnp.float32(B)), -5.0, 15.0)
        out_ref[...] = jnp.broadcast_to(out, out_ref.shape)


def kernel(z, t, e, time_landmark, time_emb, w1, b1, w2, b2, log_tau):
    B, M = z.shape
    H = w1.shape[1]
    RCE, RCH = M // NC, H // NC

    t2 = jnp.asarray(t).reshape(B, 1).astype(jnp.float32)
    erow = jnp.asarray(e).reshape(1, B).astype(jnp.float32)
    lm2 = jnp.asarray(time_landmark).reshape(1, M).astype(jnp.float32)
    emb = jnp.asarray(time_emb).astype(jnp.float32)
    w1f = jnp.asarray(w1).astype(jnp.float32)
    w2f = jnp.asarray(w2).astype(jnp.float32)
    b1f = jnp.asarray(b1).reshape(1, H).astype(jnp.float32)
    b2f = jnp.asarray(b2).reshape(1, M).astype(jnp.float32)
    logtau2 = jnp.asarray(log_tau).reshape(1, 1).astype(jnp.float32)

    out_wide = pl.pallas_call(
        functools.partial(_fused_kernel, M=M, B=B, RCE=RCE),
        out_shape=jax.ShapeDtypeStruct((B, OUT_LANES), jnp.float32),
        grid=(NC,),
        in_specs=[
            pl.BlockSpec((B, 1), lambda k: (0, 0)),
            pl.BlockSpec((1, M), lambda k: (0, 0)),
            pl.BlockSpec((1, B), lambda k: (0, 0)),
            pl.BlockSpec((RCE, M), lambda k: (k, 0)),
            pl.BlockSpec((B, M), lambda k: (0, 0)),
            pl.BlockSpec((M, RCH), lambda k: (0, k)),
            pl.BlockSpec((1, RCH), lambda k: (0, k)),
            pl.BlockSpec((RCH, M), lambda k: (k, 0)),
            pl.BlockSpec((1, M), lambda k: (0, 0)),
            pl.BlockSpec((1, 1), lambda k: (0, 0)),
        ],
        out_specs=pl.BlockSpec((B, OUT_LANES), lambda k: (0, 0)),
        scratch_shapes=[
            pltpu.VMEM((B, M), jnp.float32),
            pltpu.VMEM((NC, B, RCE), jnp.float32),
            pltpu.VMEM((B, M), jnp.float32),
            pltpu.VMEM((B, M), jnp.bfloat16),
            pltpu.VMEM((B, OUT_LANES), jnp.float32),
            pltpu.VMEM((B, M), jnp.bfloat16),
            pltpu.VMEM((NC, B, RCE), jnp.bfloat16),
        ],
        compiler_params=pltpu.CompilerParams(
            dimension_semantics=("arbitrary",),
            vmem_limit_bytes=VMEM_LIMIT),
        cost_estimate=pl.CostEstimate(
            flops=int(6 * B * M * M // NC + 4 * B * M * H // NC),
            transcendentals=int(B * B + 4 * B),
            bytes_accessed=int(4 * (M * M + M * H + H * M + B * M) // NC),
    ))(t2, lm2, erow, emb, z, w1f, b1f, w2f, b2f, logtau2)

    return out_wide[:, :1]


# speedup vs baseline: 1.3696x; 1.0939x over previous
import functools

import jax
import jax.numpy as jnp
from jax.experimental import pallas as pl
from jax.experimental.pallas import tpu as pltpu

OUT_LANES = 128
VMEM_LIMIT = 60 * 1024 * 1024
NC = 4


def _l2_normalize(x, eps=1e-12):
    ss = jnp.sum(x * x, axis=-1, keepdims=True)
    return x * jax.lax.rsqrt(jnp.maximum(ss, eps * eps))


def _fused_kernel(t_ref, lm_ref, erow_ref, emb_ref, z_ref, w1_ref, b1_ref,
                  w2_ref, b2_ref, logtau_ref, out_ref,
                  ev_acc, cens_slots, enc_acc, mask_s, indx_s, z_bf,
                  wev_slots, *, M, B, RCE):
    k = pl.program_id(0)

    @pl.when(k == 0)
    def _():
        t = t_ref[...]
        lm = lm_ref[...]
        cnt = jnp.sum((lm < t).astype(jnp.int32), axis=1, keepdims=True)
        indx = jnp.where(cnt == 0, 1, cnt)
        indx = jnp.where(indx == M, M - 1, indx)
        kf = jax.lax.broadcasted_iota(jnp.int32, (B, M), 1)
        oh_i = (kf == indx).astype(jnp.float32)
        oh_im1 = (kf == (indx - 1)).astype(jnp.float32)
        lm_i = jnp.sum(oh_i * lm, axis=1, keepdims=True)
        lm_im1 = jnp.sum(oh_im1 * lm, axis=1, keepdims=True)
        s = (t - lm_im1) / (lm_i - lm_im1)
        mask_s[...] = (kf >= indx).astype(jnp.bfloat16)
        indx_s[...] = jnp.broadcast_to(indx.astype(jnp.float32),
                                       (B, OUT_LANES))
        z_bf[...] = z_ref[...].astype(jnp.bfloat16)
        w_ev = (oh_im1 * (1.0 - s) + oh_i * s).astype(jnp.bfloat16)
        for j in range(NC):
            wev_slots[j] = w_ev[:, j * RCE:(j + 1) * RCE]

    emb_bf = emb_ref[...].astype(jnp.bfloat16)

    evc = jax.lax.dot_general(wev_slots[k], emb_bf, (((1,), (0,)), ((), ())),
                              preferred_element_type=jnp.float32)

    cens_slots[k] = jax.lax.dot_general(mask_s[...], emb_bf, (((1,), (1,)), ((), ())),
                                        preferred_element_type=jnp.float32)

    h = jnp.maximum(
        jnp.dot(z_bf[...], w1_ref[...].astype(jnp.bfloat16),
                preferred_element_type=jnp.float32)
        + b1_ref[...], 0.0)
    encc = jnp.dot(h.astype(jnp.bfloat16), w2_ref[...].astype(jnp.bfloat16),
                   preferred_element_type=jnp.float32)

    @pl.when(k == 0)
    def _():
        ev_acc[...] = evc
        enc_acc[...] = encc

    @pl.when(k > 0)
    def _():
        ev_acc[...] = ev_acc[...] + evc
        enc_acc[...] = enc_acc[...] + encc

    @pl.when(k == NC - 1)
    def _():
        eye_f = (jax.lax.broadcasted_iota(jnp.int32, (B, B), 0)
                 == jax.lax.broadcasted_iota(jnp.int32, (B, B), 1)
                 ).astype(jnp.float32)

        def to_row(col):
            return jax.lax.dot_general(col, eye_f, (((0,), (0,)), ((), ())),
                                       preferred_element_type=jnp.float32)

        ev = ev_acc[...]
        ssev_row = to_row(jnp.sum(ev * ev, axis=1, keepdims=True))
        tinv_row = to_row(1.0 / (jnp.float32(M) - indx_s[:, 0:1]))

        sscn = jnp.zeros((B, 1), jnp.float32)
        for j in range(NC):
            cj = cens_slots[j]
            sscn = sscn + jnp.sum(cj * cj, axis=1, keepdims=True)
        sscn_row = to_row(sscn)

        inv_tau_sq = jnp.exp(-logtau_ref[...])
        enc = enc_acc[...] + b2_ref[...]
        hz = _l2_normalize(enc) * inv_tau_sq

        sim_ev = jax.lax.dot_general(hz, ev, (((1,), (1,)), ((), ())),
                                     preferred_element_type=jnp.float32)
        sim_cn = jnp.zeros((B, B), jnp.float32)
        for j in range(NC):
            sim_cn = sim_cn + jax.lax.dot_general(
                hz[:, j * RCE:(j + 1) * RCE], cens_slots[j],
                (((1,), (1,)), ((), ())), preferred_element_type=jnp.float32)

        e_row = erow_ref[...]
        w_cn = (1.0 - e_row) * tinv_row
        ss_row = e_row * ssev_row + w_cn * tinv_row * sscn_row
        scale = jax.lax.rsqrt(jnp.maximum(ss_row, jnp.float32(1e-24)))
        sim = (e_row * sim_ev + w_cn * sim_cn) * scale

        g = jnp.sum(sim * eye_f, axis=1, keepdims=True)
        mx = jnp.max(sim, axis=1, keepdims=True)
        lse = mx + jnp.log(jnp.sum(jnp.exp(sim - mx), axis=1, keepdims=True))
        out = jnp.clip((lse - g) - jnp.log(jnp.float32(B)), -5.0, 15.0)
        out_ref[...] = jnp.broadcast_to(out, out_ref.shape)


def kernel(z, t, e, time_landmark, time_emb, w1, b1, w2, b2, log_tau):
    B, M = z.shape
    H = w1.shape[1]
    RCE, RCH = M // NC, H // NC

    t2 = jnp.asarray(t).reshape(B, 1).astype(jnp.float32)
    erow = jnp.asarray(e).reshape(1, B).astype(jnp.float32)
    lm2 = jnp.asarray(time_landmark).reshape(1, M).astype(jnp.float32)
    emb = jnp.asarray(time_emb).astype(jnp.float32)
    w1f = jnp.asarray(w1).astype(jnp.float32)
    w2f = jnp.asarray(w2).astype(jnp.float32)
    b1f = jnp.asarray(b1).reshape(1, H).astype(jnp.float32)
    b2f = jnp.asarray(b2).reshape(1, M).astype(jnp.float32)
    logtau2 = jnp.asarray(log_tau).reshape(1, 1).astype(jnp.float32)

    out_wide = pl.pallas_call(
        functools.partial(_fused_kernel, M=M, B=B, RCE=RCE),
        out_shape=jax.ShapeDtypeStruct((B, OUT_LANES), jnp.float32),
        grid=(NC,),
        in_specs=[
            pl.BlockSpec((B, 1), lambda k: (0, 0)),
            pl.BlockSpec((1, M), lambda k: (0, 0)),
            pl.BlockSpec((1, B), lambda k: (0, 0)),
            pl.BlockSpec((RCE, M), lambda k: (k, 0)),
            pl.BlockSpec((B, M), lambda k: (0, 0)),
            pl.BlockSpec((M, RCH), lambda k: (0, k)),
            pl.BlockSpec((1, RCH), lambda k: (0, k)),
            pl.BlockSpec((RCH, M), lambda k: (k, 0)),
            pl.BlockSpec((1, M), lambda k: (0, 0)),
            pl.BlockSpec((1, 1), lambda k: (0, 0)),
        ],
        out_specs=pl.BlockSpec((B, OUT_LANES), lambda k: (0, 0)),
        scratch_shapes=[
            pltpu.VMEM((B, M), jnp.float32),
            pltpu.VMEM((NC, B, RCE), jnp.float32),
            pltpu.VMEM((B, M), jnp.float32),
            pltpu.VMEM((B, M), jnp.bfloat16),
            pltpu.VMEM((B, OUT_LANES), jnp.float32),
            pltpu.VMEM((B, M), jnp.bfloat16),
            pltpu.VMEM((NC, B, RCE), jnp.bfloat16),
        ],
        compiler_params=pltpu.CompilerParams(
            dimension_semantics=("arbitrary",),
            vmem_limit_bytes=VMEM_LIMIT),
        cost_estimate=pl.CostEstimate(
            flops=int(6 * B * M * M // NC + 4 * B * M * H // NC),
            transcendentals=int(B * B + 4 * B),
            bytes_accessed=int(4 * (M * M + M * H + H * M + B * M) // NC),
    ))(t2, lm2, erow, emb, z, w1f, b1f, w2f, b2f, logtau2)

    return out_wide[:, :1]


# speedup vs baseline: 1.3713x; 1.0013x over previous
import functools

import jax
import jax.numpy as jnp
from jax.experimental import pallas as pl
from jax.experimental.pallas import tpu as pltpu

OUT_LANES = 128
VMEM_LIMIT = 60 * 1024 * 1024
NC = 2


def _l2_normalize(x, eps=1e-12):
    ss = jnp.sum(x * x, axis=-1, keepdims=True)
    return x * jax.lax.rsqrt(jnp.maximum(ss, eps * eps))


def _fused_kernel(t_ref, lm_ref, erow_ref, emb_ref, z_ref, w1_ref, b1_ref,
                  w2_ref, b2_ref, logtau_ref, out_ref,
                  ev_acc, cens_slots, enc_acc, mask_s, indx_s, z_bf,
                  wev_slots, *, M, B, RCE):
    k = pl.program_id(0)

    @pl.when(k == 0)
    def _():
        t = t_ref[...]
        lm = lm_ref[...]
        cnt = jnp.sum((lm < t).astype(jnp.int32), axis=1, keepdims=True)
        indx = jnp.where(cnt == 0, 1, cnt)
        indx = jnp.where(indx == M, M - 1, indx)
        kf = jax.lax.broadcasted_iota(jnp.int32, (B, M), 1)
        oh_i = (kf == indx).astype(jnp.float32)
        oh_im1 = (kf == (indx - 1)).astype(jnp.float32)
        lm_i = jnp.sum(oh_i * lm, axis=1, keepdims=True)
        lm_im1 = jnp.sum(oh_im1 * lm, axis=1, keepdims=True)
        s = (t - lm_im1) / (lm_i - lm_im1)
        mask_s[...] = (kf >= indx).astype(jnp.bfloat16)
        indx_s[...] = jnp.broadcast_to(indx.astype(jnp.float32),
                                       (B, OUT_LANES))
        z_bf[...] = z_ref[...].astype(jnp.bfloat16)
        w_ev = (oh_im1 * (1.0 - s) + oh_i * s).astype(jnp.bfloat16)
        for j in range(NC):
            wev_slots[j] = w_ev[:, j * RCE:(j + 1) * RCE]

    emb_bf = emb_ref[...].astype(jnp.bfloat16)

    evc = jax.lax.dot_general(wev_slots[k], emb_bf, (((1,), (0,)), ((), ())),
                              preferred_element_type=jnp.float32)

    cens_slots[k] = jax.lax.dot_general(mask_s[...], emb_bf, (((1,), (1,)), ((), ())),
                                        preferred_element_type=jnp.float32)

    h = jnp.maximum(
        jnp.dot(z_bf[...], w1_ref[...].astype(jnp.bfloat16),
                preferred_element_type=jnp.float32)
        + b1_ref[...], 0.0)
    encc = jnp.dot(h.astype(jnp.bfloat16), w2_ref[...].astype(jnp.bfloat16),
                   preferred_element_type=jnp.float32)

    @pl.when(k == 0)
    def _():
        ev_acc[...] = evc
        enc_acc[...] = encc

    @pl.when(k > 0)
    def _():
        ev_acc[...] = ev_acc[...] + evc
        enc_acc[...] = enc_acc[...] + encc

    @pl.when(k == NC - 1)
    def _():
        eye_f = (jax.lax.broadcasted_iota(jnp.int32, (B, B), 0)
                 == jax.lax.broadcasted_iota(jnp.int32, (B, B), 1)
                 ).astype(jnp.float32)

        def to_row(col):
            return jax.lax.dot_general(col, eye_f, (((0,), (0,)), ((), ())),
                                       preferred_element_type=jnp.float32)

        ev = ev_acc[...]
        ssev_row = to_row(jnp.sum(ev * ev, axis=1, keepdims=True))
        tinv_row = to_row(1.0 / (jnp.float32(M) - indx_s[:, 0:1]))

        sscn = jnp.zeros((B, 1), jnp.float32)
        for j in range(NC):
            cj = cens_slots[j]
            sscn = sscn + jnp.sum(cj * cj, axis=1, keepdims=True)
        sscn_row = to_row(sscn)

        inv_tau_sq = jnp.exp(-logtau_ref[...])
        enc = enc_acc[...] + b2_ref[...]
        hz = _l2_normalize(enc) * inv_tau_sq

        sim_ev = jax.lax.dot_general(hz, ev, (((1,), (1,)), ((), ())),
                                     preferred_element_type=jnp.float32)
        sim_cn = jnp.zeros((B, B), jnp.float32)
        for j in range(NC):
            sim_cn = sim_cn + jax.lax.dot_general(
                hz[:, j * RCE:(j + 1) * RCE], cens_slots[j],
                (((1,), (1,)), ((), ())), preferred_element_type=jnp.float32)

        e_row = erow_ref[...]
        w_cn = (1.0 - e_row) * tinv_row
        ss_row = e_row * ssev_row + w_cn * tinv_row * sscn_row
        scale = jax.lax.rsqrt(jnp.maximum(ss_row, jnp.float32(1e-24)))
        sim = (e_row * sim_ev + w_cn * sim_cn) * scale

        g = jnp.sum(sim * eye_f, axis=1, keepdims=True)
        mx = jnp.max(sim, axis=1, keepdims=True)
        lse = mx + jnp.log(jnp.sum(jnp.exp(sim - mx), axis=1, keepdims=True))
        out = jnp.clip((lse - g) - jnp.log(jnp.float32(B)), -5.0, 15.0)
        out_ref[...] = jnp.broadcast_to(out, out_ref.shape)


def kernel(z, t, e, time_landmark, time_emb, w1, b1, w2, b2, log_tau):
    B, M = z.shape
    H = w1.shape[1]
    RCE, RCH = M // NC, H // NC

    t2 = jnp.asarray(t).reshape(B, 1).astype(jnp.float32)
    erow = jnp.asarray(e).reshape(1, B).astype(jnp.float32)
    lm2 = jnp.asarray(time_landmark).reshape(1, M).astype(jnp.float32)
    emb = jnp.asarray(time_emb).astype(jnp.float32)
    w1f = jnp.asarray(w1).astype(jnp.float32)
    w2f = jnp.asarray(w2).astype(jnp.float32)
    b1f = jnp.asarray(b1).reshape(1, H).astype(jnp.float32)
    b2f = jnp.asarray(b2).reshape(1, M).astype(jnp.float32)
    logtau2 = jnp.asarray(log_tau).reshape(1, 1).astype(jnp.float32)

    out_wide = pl.pallas_call(
        functools.partial(_fused_kernel, M=M, B=B, RCE=RCE),
        out_shape=jax.ShapeDtypeStruct((B, OUT_LANES), jnp.float32),
        grid=(NC,),
        in_specs=[
            pl.BlockSpec((B, 1), lambda k: (0, 0)),
            pl.BlockSpec((1, M), lambda k: (0, 0)),
            pl.BlockSpec((1, B), lambda k: (0, 0)),
            pl.BlockSpec((RCE, M), lambda k: (k, 0)),
            pl.BlockSpec((B, M), lambda k: (0, 0)),
            pl.BlockSpec((M, RCH), lambda k: (0, k)),
            pl.BlockSpec((1, RCH), lambda k: (0, k)),
            pl.BlockSpec((RCH, M), lambda k: (k, 0)),
            pl.BlockSpec((1, M), lambda k: (0, 0)),
            pl.BlockSpec((1, 1), lambda k: (0, 0)),
        ],
        out_specs=pl.BlockSpec((B, OUT_LANES), lambda k: (0, 0)),
        scratch_shapes=[
            pltpu.VMEM((B, M), jnp.float32),
            pltpu.VMEM((NC, B, RCE), jnp.float32),
            pltpu.VMEM((B, M), jnp.float32),
            pltpu.VMEM((B, M), jnp.bfloat16),
            pltpu.VMEM((B, OUT_LANES), jnp.float32),
            pltpu.VMEM((B, M), jnp.bfloat16),
            pltpu.VMEM((NC, B, RCE), jnp.bfloat16),
        ],
        compiler_params=pltpu.CompilerParams(
            dimension_semantics=("arbitrary",),
            vmem_limit_bytes=VMEM_LIMIT),
        cost_estimate=pl.CostEstimate(
            flops=int(6 * B * M * M // NC + 4 * B * M * H // NC),
            transcendentals=int(B * B + 4 * B),
            bytes_accessed=int(4 * (M * M + M * H + H * M + B * M) // NC),
    ))(t2, lm2, erow, emb, z, w1f, b1f, w2f, b2f, logtau2)

    return out_wide[:, :1]


# speedup vs baseline: 1.3721x; 1.0006x over previous
import functools

import jax
import jax.numpy as jnp
from jax.experimental import pallas as pl
from jax.experimental.pallas import tpu as pltpu

OUT_LANES = 128
VMEM_LIMIT = 60 * 1024 * 1024
NC = 4


def _l2_normalize(x, eps=1e-12):
    ss = jnp.sum(x * x, axis=-1, keepdims=True)
    return x * jax.lax.rsqrt(jnp.maximum(ss, eps * eps))


def _fused_kernel(t_ref, lm_ref, erow_ref, emb_ref, z_ref, w1_ref, b1_ref,
                  w2_ref, b2_ref, logtau_ref, out_ref,
                  ev_acc, cens_slots, enc_acc, mask_s, indx_s, z_bf,
                  wev_slots, *, M, B, RCE):
    k = pl.program_id(0)

    @pl.when(k == 0)
    def _():
        t = t_ref[...]
        lm = lm_ref[...]
        cnt = jnp.sum((lm < t).astype(jnp.int32), axis=1, keepdims=True)
        indx = jnp.where(cnt == 0, 1, cnt)
        indx = jnp.where(indx == M, M - 1, indx)
        kf = jax.lax.broadcasted_iota(jnp.int32, (B, M), 1)
        oh_i = (kf == indx).astype(jnp.float32)
        oh_im1 = (kf == (indx - 1)).astype(jnp.float32)
        lm_i = jnp.sum(oh_i * lm, axis=1, keepdims=True)
        lm_im1 = jnp.sum(oh_im1 * lm, axis=1, keepdims=True)
        s = (t - lm_im1) / (lm_i - lm_im1)
        mask_s[...] = (kf >= indx).astype(jnp.bfloat16)
        indx_s[...] = jnp.broadcast_to(indx.astype(jnp.float32),
                                       (B, OUT_LANES))
        z_bf[...] = z_ref[...].astype(jnp.bfloat16)
        w_ev = (oh_im1 * (1.0 - s) + oh_i * s).astype(jnp.bfloat16)
        for j in range(NC):
            wev_slots[j] = w_ev[:, j * RCE:(j + 1) * RCE]

    emb_bf = emb_ref[...].astype(jnp.bfloat16)

    evc = jax.lax.dot_general(wev_slots[k], emb_bf, (((1,), (0,)), ((), ())),
                              preferred_element_type=jnp.float32)

    cens_slots[k] = jax.lax.dot_general(mask_s[...], emb_bf, (((1,), (1,)), ((), ())),
                                        preferred_element_type=jnp.float32)

    h = jnp.maximum(
        jnp.dot(z_bf[...], w1_ref[...].astype(jnp.bfloat16),
                preferred_element_type=jnp.float32)
        + b1_ref[...], 0.0)
    encc = jnp.dot(h.astype(jnp.bfloat16), w2_ref[...].astype(jnp.bfloat16),
                   preferred_element_type=jnp.float32)

    @pl.when(k == 0)
    def _():
        ev_acc[...] = evc
        enc_acc[...] = encc

    @pl.when(k > 0)
    def _():
        ev_acc[...] = ev_acc[...] + evc
        enc_acc[...] = enc_acc[...] + encc

    @pl.when(k == NC - 1)
    def _():
        eye_f = (jax.lax.broadcasted_iota(jnp.int32, (B, B), 0)
                 == jax.lax.broadcasted_iota(jnp.int32, (B, B), 1)
                 ).astype(jnp.float32)

        def to_row(col):
            return jax.lax.dot_general(col, eye_f, (((0,), (0,)), ((), ())),
                                       preferred_element_type=jnp.float32)

        ev = ev_acc[...]
        ssev_row = to_row(jnp.sum(ev * ev, axis=1, keepdims=True))
        tinv_row = to_row(1.0 / (jnp.float32(M) - indx_s[:, 0:1]))

        sscn = jnp.zeros((B, 1), jnp.float32)
        for j in range(NC):
            cj = cens_slots[j]
            sscn = sscn + jnp.sum(cj * cj, axis=1, keepdims=True)
        sscn_row = to_row(sscn)

        inv_tau_sq = jnp.exp(-logtau_ref[...])
        enc = enc_acc[...] + b2_ref[...]
        hz = _l2_normalize(enc) * inv_tau_sq

        sim_ev = jax.lax.dot_general(hz, ev, (((1,), (1,)), ((), ())),
                                     preferred_element_type=jnp.float32)
        sim_cn = jnp.zeros((B, B), jnp.float32)
        for j in range(NC):
            sim_cn = sim_cn + jax.lax.dot_general(
                hz[:, j * RCE:(j + 1) * RCE], cens_slots[j],
                (((1,), (1,)), ((), ())), preferred_element_type=jnp.float32)

        e_row = erow_ref[...]
        w_cn = (1.0 - e_row) * tinv_row
        ss_row = e_row * ssev_row + w_cn * tinv_row * sscn_row
        scale = jax.lax.rsqrt(jnp.maximum(ss_row, jnp.float32(1e-24)))
        sim = (e_row * sim_ev + w_cn * sim_cn) * scale

        g = jnp.sum(sim * eye_f, axis=1, keepdims=True)
        mx = jnp.max(sim, axis=1, keepdims=True)
        lse = mx + jnp.log(jnp.sum(jnp.exp(sim - mx), axis=1, keepdims=True))
        out = jnp.clip((lse - g) - jnp.log(jnp.float32(B)), -5.0, 15.0)
        out_ref[...] = jnp.broadcast_to(out, out_ref.shape)


def kernel(z, t, e, time_landmark, time_emb, w1, b1, w2, b2, log_tau):
    B, M = z.shape
    H = w1.shape[1]
    RCE, RCH = M // NC, H // NC

    t2 = jnp.asarray(t).reshape(B, 1).astype(jnp.float32)
    erow = jnp.asarray(e).reshape(1, B).astype(jnp.float32)
    lm2 = jnp.asarray(time_landmark).reshape(1, M).astype(jnp.float32)
    emb = jnp.asarray(time_emb).astype(jnp.float32)
    w1f = jnp.asarray(w1).astype(jnp.float32)
    w2f = jnp.asarray(w2).astype(jnp.float32)
    b1f = jnp.asarray(b1).reshape(1, H).astype(jnp.float32)
    b2f = jnp.asarray(b2).reshape(1, M).astype(jnp.float32)
    logtau2 = jnp.asarray(log_tau).reshape(1, 1).astype(jnp.float32)

    out_wide = pl.pallas_call(
        functools.partial(_fused_kernel, M=M, B=B, RCE=RCE),
        out_shape=jax.ShapeDtypeStruct((B, OUT_LANES), jnp.float32),
        grid=(NC,),
        in_specs=[
            pl.BlockSpec((B, 1), lambda k: (0, 0)),
            pl.BlockSpec((1, M), lambda k: (0, 0)),
            pl.BlockSpec((1, B), lambda k: (0, 0)),
            pl.BlockSpec((RCE, M), lambda k: (k, 0)),
            pl.BlockSpec((B, M), lambda k: (0, 0)),
            pl.BlockSpec((M, RCH), lambda k: (0, k)),
            pl.BlockSpec((1, RCH), lambda k: (0, k)),
            pl.BlockSpec((RCH, M), lambda k: (k, 0)),
            pl.BlockSpec((1, M), lambda k: (0, 0)),
            pl.BlockSpec((1, 1), lambda k: (0, 0)),
        ],
        out_specs=pl.BlockSpec((B, OUT_LANES), lambda k: (0, 0)),
        scratch_shapes=[
            pltpu.VMEM((B, M), jnp.float32),
            pltpu.VMEM((NC, B, RCE), jnp.float32),
            pltpu.VMEM((B, M), jnp.float32),
            pltpu.VMEM((B, M), jnp.bfloat16),
            pltpu.VMEM((B, OUT_LANES), jnp.float32),
            pltpu.VMEM((B, M), jnp.bfloat16),
            pltpu.VMEM((NC, B, RCE), jnp.bfloat16),
        ],
        compiler_params=pltpu.CompilerParams(
            dimension_semantics=("arbitrary",),
            vmem_limit_bytes=VMEM_LIMIT),
        cost_estimate=pl.CostEstimate(
            flops=int(6 * B * M * M // NC + 4 * B * M * H // NC),
            transcendentals=int(B * B + 4 * B),
            bytes_accessed=int(4 * (M * M + M * H + H * M + B * M) // NC),
    ))(t2, lm2, erow, emb, z, w1f, b1f, w2f, b2f, logtau2)

    return out_wide[:, :1]
